# baseline (speedup 1.0000x reference)
import jax
import jax.numpy as jnp
from jax.experimental import pallas as pl

N_NODES = 10000
D_IN = 128
D_OUT = 128
BM = 400


def _gcn_block_kernel(a_ref, x_ref, wt_ref, b_ref, o_ref):
    support = jnp.dot(a_ref[...], x_ref[...], preferred_element_type=jnp.float32)
    o_ref[...] = (
        jnp.dot(support, wt_ref[...], preferred_element_type=jnp.float32)
        + b_ref[...]
    )


def kernel(x, adj_normalized, W, b):
    wt = W.T
    b2 = b.reshape(1, D_OUT)
    grid = (N_NODES // BM,)
    out = pl.pallas_call(
        _gcn_block_kernel,
        grid=grid,
        in_specs=[
            pl.BlockSpec((BM, N_NODES), lambda i: (i, 0)),
            pl.BlockSpec((N_NODES, D_IN), lambda i: (0, 0)),
            pl.BlockSpec((D_IN, D_OUT), lambda i: (0, 0)),
            pl.BlockSpec((1, D_OUT), lambda i: (0, 0)),
        ],
        out_specs=pl.BlockSpec((BM, D_OUT), lambda i: (i, 0)),
        out_shape=jax.ShapeDtypeStruct((N_NODES, D_OUT), jnp.float32),
    )(adj_normalized, x, wt, b2)
    return out
